# Initial kernel scaffold; baseline (speedup 1.0000x reference)
import jax
import jax.numpy as jnp
from jax.experimental import pallas as pl


def kernel(ei, seq, table, W1, b1, W2, b2):
    raise NotImplementedError("write your pallas kernel here")


# speedup vs baseline: 8.3775x; 8.3775x over previous
import functools

import jax
import jax.numpy as jnp
from jax import lax
from jax.experimental import pallas as pl
from jax.experimental.pallas import tpu as pltpu
from jax.experimental.pallas import tpu_sc as plsc

N = 10000
H = 128
OFF = 2
E = 320000
NC, NS = 2, 16
NW = NC * NS
NPAD = 10240
RPT = NPAD // NS
CH = 128
EPT = 10240
K = EPT // CH
EPAD = EPT * NW
SPT = 8192 // NW


def _fill2d(ref, rows, val):
    cols = ref.shape[1] // 16
    v = jnp.full((16,), val, jnp.float32)

    def body(i, _):
        for c in range(cols):
            ref[i, pl.ds(c * 16, 16)] = v
        return 0

    lax.fori_loop(0, rows, body, 0)


@functools.cache
def _sc_kernels():
    mesh = plsc.VectorSubcoreMesh(
        core_axis_name="c", subcore_axis_name="s",
        num_cores=NC, num_subcores=NS)

    @functools.partial(
        pl.kernel,
        out_type=jax.ShapeDtypeStruct((NW, NPAD), jnp.float32),
        mesh=mesh,
        scratch_types=[
            pltpu.VMEM((K, CH), jnp.int32),
            pltpu.VMEM((NPAD,), jnp.float32),
        ],
        compiler_params=pltpu.CompilerParams(needs_layout_passes=False),
    )
    def deg_kernel(dst_hbm, deg_hbm, dst_v, hist):
        c = lax.axis_index("c")
        s = lax.axis_index("s")
        wid = s * NC + c
        zero = jnp.zeros((16,), jnp.float32)
        one = jnp.ones((16,), jnp.float32)

        def zbody(i, _):
            hist[pl.ds(i * 16, 16)] = zero
            return 0

        lax.fori_loop(0, NPAD // 16, zbody, 0)
        pltpu.sync_copy(dst_hbm.at[wid], dst_v)

        def body(j, _):
            for c8 in range(CH // 16):
                idx = dst_v[j, pl.ds(c8 * 16, 16)]
                plsc.addupdate_scatter(hist, [idx], one)
            return 0

        lax.fori_loop(0, K, body, 0)
        pltpu.sync_copy(hist, deg_hbm.at[wid])

    HH = H // 2

    @functools.partial(
        pl.kernel,
        out_type=jax.ShapeDtypeStruct((NC, 2, NPAD, HH), jnp.float32),
        mesh=mesh,
        scratch_types=[
            pltpu.VMEM((K, CH), jnp.int32),
            pltpu.VMEM((K, CH), jnp.int32),
            pltpu.VMEM((CH, HH), jnp.float32),
            pltpu.VMEM((CH, HH), jnp.float32),
            pltpu.VMEM((CH, HH), jnp.float32),
            pltpu.VMEM_SHARED((NPAD, HH), jnp.float32),
            pltpu.SemaphoreType.DMA,
            pltpu.SemaphoreType.DMA,
        ],
        compiler_params=pltpu.CompilerParams(use_tc_tiling_on_sc=False),
    )
    def agg_kernel(hs_hbm, src_hbm, dst_hbm, part_hbm,
                   src_v, dst_v, buf_a, buf_b, zbuf, acc, sem_a, sem_b):
        c = lax.axis_index("c")
        s = lax.axis_index("s")
        wid = s * NC + c
        _fill2d(zbuf, CH, 0.0)
        pltpu.sync_copy(src_hbm.at[wid], src_v)
        pltpu.sync_copy(dst_hbm.at[wid], dst_v)

        for f in range(2):
            half = hs_hbm.at[f]
            for r in range(RPT // CH):
                pltpu.sync_copy(zbuf, acc.at[pl.ds(s * RPT + r * CH, CH)])
            plsc.subcore_barrier()

            pltpu.async_copy(half.at[src_v.at[0]], buf_a, sem_a)

            def body(k, _):
                j = 2 * k
                pltpu.make_async_copy(
                    half.at[src_v.at[j]], buf_a, sem_a).wait()
                pltpu.async_copy(half.at[src_v.at[j + 1]], buf_b, sem_b)
                pltpu.sync_copy(buf_a, acc.at[dst_v.at[j]], add=True)
                pltpu.make_async_copy(
                    half.at[src_v.at[j + 1]], buf_b, sem_b).wait()

                @pl.when(k < K // 2 - 1)
                def _():
                    pltpu.async_copy(half.at[src_v.at[j + 2]], buf_a, sem_a)

                pltpu.sync_copy(buf_b, acc.at[dst_v.at[j + 1]], add=True)
                return 0

            lax.fori_loop(0, K // 2, body, 0)
            plsc.subcore_barrier()
            pltpu.sync_copy(acc.at[pl.ds(s * RPT, RPT)],
                            part_hbm.at[c, f, pl.ds(s * RPT, RPT)])

    @functools.partial(
        pl.kernel,
        out_type=jax.ShapeDtypeStruct((8192, H), jnp.float32),
        mesh=mesh,
        scratch_types=[
            pltpu.VMEM((SPT // CH, CH), jnp.int32),
            pltpu.VMEM((CH, H), jnp.float32),
            pltpu.VMEM((CH, H), jnp.float32),
            pltpu.SemaphoreType.DMA,
            pltpu.SemaphoreType.DMA,
        ],
    )
    def seq_kernel(ztab_hbm, idx_hbm, out_hbm,
                   idx_v, buf_a, buf_b, sem_a, sem_b):
        c = lax.axis_index("c")
        s = lax.axis_index("s")
        wid = s * NC + c
        pltpu.sync_copy(idx_hbm.at[wid], idx_v)
        pltpu.async_copy(ztab_hbm.at[idx_v.at[0]], buf_a, sem_a)
        pltpu.async_copy(ztab_hbm.at[idx_v.at[1]], buf_b, sem_b)
        pltpu.make_async_copy(ztab_hbm.at[idx_v.at[0]], buf_a, sem_a).wait()
        pltpu.sync_copy(buf_a, out_hbm.at[pl.ds(wid * SPT, CH)])
        pltpu.make_async_copy(ztab_hbm.at[idx_v.at[1]], buf_b, sem_b).wait()
        pltpu.sync_copy(buf_b, out_hbm.at[pl.ds(wid * SPT + CH, CH)])

    return deg_kernel, agg_kernel, seq_kernel


RB = 1024
GRID = NPAD // RB


def _dinv_of(deg_col):
    return lax.rsqrt(deg_col + 1.0)


HH = H // 2


def _split_store(out_ref, hs):
    out_ref[0] = hs[:, :HH]
    out_ref[1] = hs[:, HH:]


def _prep_body(x_ref, w_ref, deg_ref, out_ref):
    dinv = _dinv_of(deg_ref[...])
    h = jnp.dot(x_ref[...], w_ref[...], preferred_element_type=jnp.float32)
    _split_store(out_ref, h * dinv)


def _mid_body(p_ref, hs_ref, deg_ref, b_ref, w_ref, a_ref, hs_out_ref):
    dinv = _dinv_of(deg_ref[...])
    agg = jnp.concatenate(
        [p_ref[0, 0] + p_ref[1, 0] + hs_ref[0],
         p_ref[0, 1] + p_ref[1, 1] + hs_ref[1]], axis=1)
    a = agg * dinv + b_ref[...]
    a_ref[...] = a
    h2 = jnp.dot(jnp.maximum(a, 0.0), w_ref[...],
                 preferred_element_type=jnp.float32)
    _split_store(hs_out_ref, h2 * dinv)


_row_spec = pl.BlockSpec((RB, H), lambda i: (i, 0))
_deg_spec = pl.BlockSpec((RB, 1), lambda i: (i, 0))
_w_spec = pl.BlockSpec((H, H), lambda i: (0, 0))
_b_spec = pl.BlockSpec((1, H), lambda i: (0, 0))
_hs_spec = pl.BlockSpec((2, RB, HH), lambda i: (0, i, 0))
_p_spec = pl.BlockSpec((NC, 2, RB, HH), lambda i: (0, 0, i, 0))
_vec_out = jax.ShapeDtypeStruct((NPAD, H), jnp.float32)
_hs_out = jax.ShapeDtypeStruct((2, NPAD, HH), jnp.float32)

_prep_call = pl.pallas_call(
    _prep_body, grid=(GRID,), out_shape=_hs_out,
    in_specs=[_row_spec, _w_spec, _deg_spec], out_specs=_hs_spec)

_mid_call = pl.pallas_call(
    _mid_body, grid=(GRID,), out_shape=(_vec_out, _hs_out),
    in_specs=[_p_spec, _hs_spec, _deg_spec, _b_spec, _w_spec],
    out_specs=(_row_spec, _hs_spec))


@jax.jit
def _run(ei, seq, table, W1, b1, W2, b2):
    deg_kernel, agg_kernel, seq_kernel = _sc_kernels()
    src = ei[0].astype(jnp.int32)
    dst = ei[1].astype(jnp.int32)
    pad = EPAD - E
    src_p = jnp.concatenate([src, jnp.zeros((pad,), jnp.int32)])
    dst_p = jnp.concatenate([dst, jnp.full((pad,), N, jnp.int32)])
    src_p = src_p.reshape(NW, K, CH)
    dst_p = dst_p.reshape(NW, K, CH)

    x_pad = jnp.concatenate(
        [table[:N], jnp.zeros((NPAD - N, H), jnp.float32)])

    deg32 = deg_kernel(dst_p)
    deg2 = deg32.sum(axis=0).reshape(NPAD, 1)

    bstack = jnp.stack([b1.reshape(1, H), b2.reshape(1, H)])

    hs = _prep_call(x_pad, W1, deg2)

    def layer(i, carry):
        hs, _ = carry
        p = agg_kernel(hs, src_p, dst_p)
        b_i = lax.dynamic_index_in_dim(bstack, i, 0, keepdims=False)
        a, hs_next = _mid_call(p, hs, deg2, b_i, W2)
        return hs_next, a

    _, z = lax.fori_loop(
        0, 2, layer, (hs, jnp.zeros((NPAD, H), jnp.float32)))

    ztab = jnp.concatenate([z[:N], table[N:N + OFF]])
    sidx = seq.astype(jnp.int32)
    idx = jnp.where(sidx >= 0, sidx, sidx + OFF + N)
    idx = idx.reshape(NW, SPT // CH, CH)
    embs = seq_kernel(ztab, idx)
    return embs.reshape(seq.shape[0], seq.shape[1], H)


def kernel(ei, seq, table, W1, b1, W2, b2):
    return _run(ei, seq, table, W1, b1, W2, b2)


# speedup vs baseline: 8.5449x; 1.0200x over previous
import functools

import jax
import jax.numpy as jnp
from jax import lax
from jax.experimental import pallas as pl
from jax.experimental.pallas import tpu as pltpu
from jax.experimental.pallas import tpu_sc as plsc

N = 10000
H = 128
OFF = 2
E = 320000
NC, NS = 2, 16
NW = NC * NS
NPAD = 10240
RPT = NPAD // NS
CH = 128
EPT = 10240
K = EPT // CH
EPAD = EPT * NW
SPT = 8192 // NW
HH = 32


def _fill2d(ref, rows, val):
    cols = ref.shape[1] // 16
    v = jnp.full((16,), val, jnp.float32)

    def body(i, _):
        for c in range(cols):
            ref[i, pl.ds(c * 16, 16)] = v
        return 0

    lax.fori_loop(0, rows, body, 0)


@functools.cache
def _sc_kernels():
    mesh = plsc.VectorSubcoreMesh(
        core_axis_name="c", subcore_axis_name="s",
        num_cores=NC, num_subcores=NS)

    @functools.partial(
        pl.kernel,
        out_type=jax.ShapeDtypeStruct((NW, NPAD), jnp.float32),
        mesh=mesh,
        scratch_types=[
            pltpu.VMEM((K, CH), jnp.int32),
            pltpu.VMEM((NPAD,), jnp.float32),
        ],
        compiler_params=pltpu.CompilerParams(needs_layout_passes=False),
    )
    def deg_kernel(dst_hbm, deg_hbm, dst_v, hist):
        c = lax.axis_index("c")
        s = lax.axis_index("s")
        wid = s * NC + c
        zero = jnp.zeros((16,), jnp.float32)
        one = jnp.ones((16,), jnp.float32)

        def zbody(i, _):
            hist[pl.ds(i * 16, 16)] = zero
            return 0

        lax.fori_loop(0, NPAD // 16, zbody, 0)
        pltpu.sync_copy(dst_hbm.at[wid], dst_v)

        def body(j, _):
            for c8 in range(CH // 16):
                idx = dst_v[j, pl.ds(c8 * 16, 16)]
                plsc.addupdate_scatter(hist, [idx], one)
            return 0

        lax.fori_loop(0, K, body, 0)
        pltpu.sync_copy(hist, deg_hbm.at[wid])

    NF = H // HH

    @functools.partial(
        pl.kernel,
        out_type=jax.ShapeDtypeStruct((NC, NF, NPAD, HH), jnp.float32),
        mesh=mesh,
        scratch_types=[
            pltpu.VMEM((K, CH), jnp.int32),
            pltpu.VMEM((K, CH), jnp.int32),
            pltpu.VMEM((8, CH, HH), jnp.float32),
            pltpu.VMEM((CH, HH), jnp.float32),
            pltpu.VMEM_SHARED((NPAD, HH), jnp.float32),
            [pltpu.SemaphoreType.DMA] * 8,
            [pltpu.SemaphoreType.DMA] * 8,
        ],
        compiler_params=pltpu.CompilerParams(use_tc_tiling_on_sc=False),
    )
    def agg_kernel(hs_hbm, src_hbm, dst_hbm, part_hbm,
                   src_v, dst_v, bufs, zbuf, acc, gsems, ssems):
        c = lax.axis_index("c")
        s = lax.axis_index("s")
        wid = s * NC + c
        _fill2d(zbuf, CH, 0.0)
        pltpu.sync_copy(src_hbm.at[wid], src_v)
        pltpu.sync_copy(dst_hbm.at[wid], dst_v)

        NQ = K // 4

        for f in range(NF):
            half = hs_hbm.at[f]
            for r in range(RPT // CH):
                pltpu.sync_copy(zbuf, acc.at[pl.ds(s * RPT + r * CH, CH)])
            plsc.subcore_barrier()

            for b in range(4):
                pltpu.async_copy(
                    half.at[src_v.at[b]], bufs.at[b], gsems[b])
            for b in range(4):
                pltpu.async_copy(
                    half.at[src_v.at[4 + b]], bufs.at[4 + b], gsems[4 + b])

            def body(qp, _):
                for hq in range(2):
                    q = 2 * qp + hq
                    aset = 4 * hq
                    j0 = 4 * q
                    for b in range(4):
                        j = j0 + b
                        slot = aset + b
                        pltpu.make_async_copy(
                            half.at[src_v.at[j]], bufs.at[slot],
                            gsems[slot]).wait()
                        pltpu.async_copy(
                            bufs.at[slot], acc.at[dst_v.at[j]],
                            ssems[slot], add=True)
                    @pl.when(q < NQ - 2)
                    def _():
                        for b in range(4):
                            j = j0 + b
                            slot = aset + b
                            pltpu.make_async_copy(
                                bufs.at[slot], acc.at[dst_v.at[j]],
                                ssems[slot]).wait()
                            pltpu.async_copy(
                                half.at[src_v.at[j + 8]], bufs.at[slot],
                                gsems[slot])
                return 0

            lax.fori_loop(0, NQ // 2, body, 0)
            for qq in (NQ - 2, NQ - 1):
                for b in range(4):
                    j = 4 * qq + b
                    slot = (qq % 2) * 4 + b
                    pltpu.make_async_copy(
                        bufs.at[slot], acc.at[dst_v.at[j]],
                        ssems[slot]).wait()
            plsc.subcore_barrier()
            pltpu.sync_copy(acc.at[pl.ds(s * RPT, RPT)],
                            part_hbm.at[c, f, pl.ds(s * RPT, RPT)])

    @functools.partial(
        pl.kernel,
        out_type=jax.ShapeDtypeStruct((8192, H), jnp.float32),
        mesh=mesh,
        scratch_types=[
            pltpu.VMEM((SPT // CH, CH), jnp.int32),
            pltpu.VMEM((CH, H), jnp.float32),
            pltpu.VMEM((CH, H), jnp.float32),
            pltpu.SemaphoreType.DMA,
            pltpu.SemaphoreType.DMA,
        ],
    )
    def seq_kernel(ztab_hbm, idx_hbm, out_hbm,
                   idx_v, buf_a, buf_b, sem_a, sem_b):
        c = lax.axis_index("c")
        s = lax.axis_index("s")
        wid = s * NC + c
        pltpu.sync_copy(idx_hbm.at[wid], idx_v)
        pltpu.async_copy(ztab_hbm.at[idx_v.at[0]], buf_a, sem_a)
        pltpu.async_copy(ztab_hbm.at[idx_v.at[1]], buf_b, sem_b)
        pltpu.make_async_copy(ztab_hbm.at[idx_v.at[0]], buf_a, sem_a).wait()
        pltpu.sync_copy(buf_a, out_hbm.at[pl.ds(wid * SPT, CH)])
        pltpu.make_async_copy(ztab_hbm.at[idx_v.at[1]], buf_b, sem_b).wait()
        pltpu.sync_copy(buf_b, out_hbm.at[pl.ds(wid * SPT + CH, CH)])

    return deg_kernel, agg_kernel, seq_kernel


RB = 1024
GRID = NPAD // RB


def _dinv_of(deg_col):
    return lax.rsqrt(deg_col + 1.0)


NF = H // HH


def _split_store(out_ref, hs):
    for i in range(NF):
        out_ref[i] = hs[:, i * HH:(i + 1) * HH]


def _prep_body(x_ref, w_ref, deg_ref, out_ref):
    dinv = _dinv_of(deg_ref[...])
    h = jnp.dot(x_ref[...], w_ref[...], preferred_element_type=jnp.float32)
    _split_store(out_ref, h * dinv)


def _mid_body(p_ref, hs_ref, deg_ref, b_ref, w_ref, a_ref, hs_out_ref):
    dinv = _dinv_of(deg_ref[...])
    agg = jnp.concatenate(
        [p_ref[0, i] + p_ref[1, i] + hs_ref[i] for i in range(NF)], axis=1)
    a = agg * dinv + b_ref[...]
    a_ref[...] = a
    h2 = jnp.dot(jnp.maximum(a, 0.0), w_ref[...],
                 preferred_element_type=jnp.float32)
    _split_store(hs_out_ref, h2 * dinv)


_row_spec = pl.BlockSpec((RB, H), lambda i: (i, 0))
_deg_spec = pl.BlockSpec((RB, 1), lambda i: (i, 0))
_w_spec = pl.BlockSpec((H, H), lambda i: (0, 0))
_b_spec = pl.BlockSpec((1, H), lambda i: (0, 0))
_hs_spec = pl.BlockSpec((NF, RB, HH), lambda i: (0, i, 0))
_p_spec = pl.BlockSpec((NC, NF, RB, HH), lambda i: (0, 0, i, 0))
_vec_out = jax.ShapeDtypeStruct((NPAD, H), jnp.float32)
_hs_out = jax.ShapeDtypeStruct((NF, NPAD, HH), jnp.float32)

_prep_call = pl.pallas_call(
    _prep_body, grid=(GRID,), out_shape=_hs_out,
    in_specs=[_row_spec, _w_spec, _deg_spec], out_specs=_hs_spec)

_mid_call = pl.pallas_call(
    _mid_body, grid=(GRID,), out_shape=(_vec_out, _hs_out),
    in_specs=[_p_spec, _hs_spec, _deg_spec, _b_spec, _w_spec],
    out_specs=(_row_spec, _hs_spec))


@jax.jit
def _run(ei, seq, table, W1, b1, W2, b2):
    deg_kernel, agg_kernel, seq_kernel = _sc_kernels()
    src = ei[0].astype(jnp.int32)
    dst = ei[1].astype(jnp.int32)
    pad = EPAD - E
    src_p = jnp.concatenate([src, jnp.zeros((pad,), jnp.int32)])
    dst_p = jnp.concatenate([dst, jnp.full((pad,), N, jnp.int32)])
    src_p = src_p.reshape(NW, K, CH)
    dst_p = dst_p.reshape(NW, K, CH)

    x_pad = jnp.concatenate(
        [table[:N], jnp.zeros((NPAD - N, H), jnp.float32)])

    deg32 = deg_kernel(dst_p)
    deg2 = deg32.sum(axis=0).reshape(NPAD, 1)

    bstack = jnp.stack([b1.reshape(1, H), b2.reshape(1, H)])

    hs = _prep_call(x_pad, W1, deg2)

    p1 = agg_kernel(hs, src_p, dst_p)
    _, hs2 = _mid_call(p1, hs, deg2, bstack[0], W2)
    p2 = agg_kernel(hs2, src_p, dst_p)
    z, _ = _mid_call(p2, hs2, deg2, bstack[1], W2)

    ztab = jnp.concatenate([z[:N], table[N:N + OFF]])
    sidx = seq.astype(jnp.int32)
    idx = jnp.where(sidx >= 0, sidx, sidx + OFF + N)
    idx = idx.reshape(NW, SPT // CH, CH)
    embs = seq_kernel(ztab, idx)
    return embs.reshape(seq.shape[0], seq.shape[1], H)


def kernel(ei, seq, table, W1, b1, W2, b2):
    return _run(ei, seq, table, W1, b1, W2, b2)


# speedup vs baseline: 22.1459x; 2.5917x over previous
import functools

import jax
import jax.numpy as jnp
from jax import lax
from jax.experimental import pallas as pl
from jax.experimental.pallas import tpu as pltpu
from jax.experimental.pallas import tpu_sc as plsc

N = 10000
H = 128
OFF = 2
E = 320000
NC, NS = 2, 16
NW = NC * NS
NPAD = 10240
RPT = NPAD // NS
CH = 128
EPT = 10240
K = EPT // CH
EPAD = EPT * NW
SPT = 8192 // NW
HH = 32


def _fill2d(ref, rows, val):
    cols = ref.shape[1] // 16
    v = jnp.full((16,), val, jnp.float32)

    def body(i, _):
        for c in range(cols):
            ref[i, pl.ds(c * 16, 16)] = v
        return 0

    lax.fori_loop(0, rows, body, 0)


@functools.cache
def _sc_kernels():
    mesh = plsc.VectorSubcoreMesh(
        core_axis_name="c", subcore_axis_name="s",
        num_cores=NC, num_subcores=NS)

    @functools.partial(
        pl.kernel,
        out_type=jax.ShapeDtypeStruct((NW, NPAD), jnp.float32),
        mesh=mesh,
        scratch_types=[
            pltpu.VMEM((K, CH), jnp.int32),
            pltpu.VMEM((NPAD,), jnp.float32),
        ],
        compiler_params=pltpu.CompilerParams(needs_layout_passes=False),
    )
    def deg_kernel(dst_hbm, deg_hbm, dst_v, hist):
        c = lax.axis_index("c")
        s = lax.axis_index("s")
        wid = s * NC + c
        zero = jnp.zeros((16,), jnp.float32)
        one = jnp.ones((16,), jnp.float32)

        def zbody(i, _):
            hist[pl.ds(i * 16, 16)] = zero
            return 0

        lax.fori_loop(0, NPAD // 16, zbody, 0)
        pltpu.sync_copy(dst_hbm.at[wid], dst_v)

        def body(j, _):
            for c8 in range(CH // 16):
                idx = dst_v[j, pl.ds(c8 * 16, 16)]
                plsc.addupdate_scatter(hist, [idx], one)
            return 0

        lax.fori_loop(0, K, body, 0)
        pltpu.sync_copy(hist, deg_hbm.at[wid])

    NF = H // HH

    @functools.partial(
        pl.kernel,
        out_type=jax.ShapeDtypeStruct((NC, NF, NPAD, HH), jnp.float32),
        mesh=mesh,
        scratch_types=[
            pltpu.VMEM((K, CH), jnp.int32),
            pltpu.VMEM((K, CH), jnp.int32),
            pltpu.VMEM((8, CH, HH), jnp.float32),
            pltpu.VMEM((CH, HH), jnp.float32),
            pltpu.VMEM_SHARED((NPAD, HH), jnp.float32),
            [pltpu.SemaphoreType.DMA] * 8,
            [pltpu.SemaphoreType.DMA] * 8,
        ],
        compiler_params=pltpu.CompilerParams(use_tc_tiling_on_sc=False),
    )
    def agg_kernel(hs_hbm, src_hbm, dst_hbm, part_hbm,
                   src_v, dst_v, bufs, zbuf, acc, gsems, ssems):
        c = lax.axis_index("c")
        s = lax.axis_index("s")
        wid = s * NC + c
        _fill2d(zbuf, CH, 0.0)
        pltpu.sync_copy(src_hbm.at[wid], src_v)
        pltpu.sync_copy(dst_hbm.at[wid], dst_v)

        NQ = K // 4

        for f in range(NF):
            half = hs_hbm.at[f]
            for r in range(RPT // CH):
                pltpu.sync_copy(zbuf, acc.at[pl.ds(s * RPT + r * CH, CH)])
            plsc.subcore_barrier()

            for b in range(4):
                pltpu.async_copy(
                    half.at[src_v.at[b]], bufs.at[b], gsems[b])
            for b in range(4):
                pltpu.async_copy(
                    half.at[src_v.at[4 + b]], bufs.at[4 + b], gsems[4 + b])

            def body(qp, _):
                for hq in range(2):
                    q = 2 * qp + hq
                    aset = 4 * hq
                    j0 = 4 * q
                    for b in range(4):
                        j = j0 + b
                        slot = aset + b
                        pltpu.make_async_copy(
                            half.at[src_v.at[j]], bufs.at[slot],
                            gsems[slot]).wait()
                        pltpu.async_copy(
                            bufs.at[slot], acc.at[dst_v.at[j]],
                            ssems[slot], add=True)
                    @pl.when(q < NQ - 2)
                    def _():
                        for b in range(4):
                            j = j0 + b
                            slot = aset + b
                            pltpu.make_async_copy(
                                bufs.at[slot], acc.at[dst_v.at[j]],
                                ssems[slot]).wait()
                            pltpu.async_copy(
                                half.at[src_v.at[j + 8]], bufs.at[slot],
                                gsems[slot])
                return 0

            lax.fori_loop(0, NQ // 2, body, 0)
            for qq in (NQ - 2, NQ - 1):
                for b in range(4):
                    j = 4 * qq + b
                    slot = (qq % 2) * 4 + b
                    pltpu.make_async_copy(
                        bufs.at[slot], acc.at[dst_v.at[j]],
                        ssems[slot]).wait()
            plsc.subcore_barrier()
            pltpu.sync_copy(acc.at[pl.ds(s * RPT, RPT)],
                            part_hbm.at[c, f, pl.ds(s * RPT, RPT)])

    @functools.partial(
        pl.kernel,
        out_type=jax.ShapeDtypeStruct((8192, H), jnp.float32),
        mesh=mesh,
        scratch_types=[
            pltpu.VMEM((SPT // CH, CH), jnp.int32),
            pltpu.VMEM((CH, H), jnp.float32),
            pltpu.VMEM((CH, H), jnp.float32),
            pltpu.SemaphoreType.DMA,
            pltpu.SemaphoreType.DMA,
        ],
    )
    def seq_kernel(ztab_hbm, idx_hbm, out_hbm,
                   idx_v, buf_a, buf_b, sem_a, sem_b):
        c = lax.axis_index("c")
        s = lax.axis_index("s")
        wid = s * NC + c
        pltpu.sync_copy(idx_hbm.at[wid], idx_v)
        pltpu.async_copy(ztab_hbm.at[idx_v.at[0]], buf_a, sem_a)
        pltpu.async_copy(ztab_hbm.at[idx_v.at[1]], buf_b, sem_b)
        pltpu.make_async_copy(ztab_hbm.at[idx_v.at[0]], buf_a, sem_a).wait()
        pltpu.sync_copy(buf_a, out_hbm.at[pl.ds(wid * SPT, CH)])
        pltpu.make_async_copy(ztab_hbm.at[idx_v.at[1]], buf_b, sem_b).wait()
        pltpu.sync_copy(buf_b, out_hbm.at[pl.ds(wid * SPT + CH, CH)])

    return deg_kernel, agg_kernel, seq_kernel


RB = 1024
GRID = NPAD // RB


def _dinv_of(deg_col):
    return lax.rsqrt(deg_col + 1.0)


NF = H // HH


def _split_store(out_ref, hs):
    for i in range(NF):
        out_ref[i] = hs[:, i * HH:(i + 1) * HH]


def _prep_body(x_ref, w_ref, deg_ref, out_ref):
    dinv = _dinv_of(deg_ref[...])
    h = jnp.dot(x_ref[...], w_ref[...], preferred_element_type=jnp.float32)
    _split_store(out_ref, h * dinv)


def _mid_body(p_ref, hs_ref, deg_ref, b_ref, w_ref, a_ref, hs_out_ref):
    dinv = _dinv_of(deg_ref[...])
    agg = jnp.concatenate(
        [p_ref[0, i] + p_ref[1, i] + hs_ref[i] for i in range(NF)], axis=1)
    a = agg * dinv + b_ref[...]
    a_ref[...] = a
    h2 = jnp.dot(jnp.maximum(a, 0.0), w_ref[...],
                 preferred_element_type=jnp.float32)
    _split_store(hs_out_ref, h2 * dinv)


_row_spec = pl.BlockSpec((RB, H), lambda i: (i, 0))
_deg_spec = pl.BlockSpec((RB, 1), lambda i: (i, 0))
_w_spec = pl.BlockSpec((H, H), lambda i: (0, 0))
_b_spec = pl.BlockSpec((1, H), lambda i: (0, 0))
_hs_spec = pl.BlockSpec((NF, RB, HH), lambda i: (0, i, 0))
_p_spec = pl.BlockSpec((NC, NF, RB, HH), lambda i: (0, 0, i, 0))
_vec_out = jax.ShapeDtypeStruct((NPAD, H), jnp.float32)
_hs_out = jax.ShapeDtypeStruct((NF, NPAD, HH), jnp.float32)

_prep_call = pl.pallas_call(
    _prep_body, grid=(GRID,), out_shape=_hs_out,
    in_specs=[_row_spec, _w_spec, _deg_spec], out_specs=_hs_spec)

_mid_call = pl.pallas_call(
    _mid_body, grid=(GRID,), out_shape=(_vec_out, _hs_out),
    in_specs=[_p_spec, _hs_spec, _deg_spec, _b_spec, _w_spec],
    out_specs=(_row_spec, _hs_spec))


@jax.jit
def _run(ei, seq, table, W1, b1, W2, b2):
    deg_kernel, agg_kernel, seq_kernel = _sc_kernels()
    src = ei[0].astype(jnp.int32)
    dst = ei[1].astype(jnp.int32)
    pad = EPAD - E
    spread = N + jnp.arange(pad, dtype=jnp.int32) % (NPAD - N)
    src_p = jnp.concatenate([src, spread])
    dst_p = jnp.concatenate([dst, spread])
    src_p = src_p.reshape(NW, K, CH)
    dst_p = dst_p.reshape(NW, K, CH)

    x_pad = jnp.concatenate(
        [table[:N], jnp.zeros((NPAD - N, H), jnp.float32)])

    deg32 = deg_kernel(dst_p)
    deg2 = deg32.sum(axis=0).reshape(NPAD, 1)

    bstack = jnp.stack([b1.reshape(1, H), b2.reshape(1, H)])

    hs = _prep_call(x_pad, W1, deg2)

    p1 = agg_kernel(hs, src_p, dst_p)
    _, hs2 = _mid_call(p1, hs, deg2, bstack[0], W2)
    p2 = agg_kernel(hs2, src_p, dst_p)
    z, _ = _mid_call(p2, hs2, deg2, bstack[1], W2)

    ztab = jnp.concatenate([z[:N], table[N:N + OFF]])
    sidx = seq.astype(jnp.int32)
    idx = jnp.where(sidx >= 0, sidx, sidx + OFF + N)
    idx = idx.reshape(NW, SPT // CH, CH)
    embs = seq_kernel(ztab, idx)
    return embs.reshape(seq.shape[0], seq.shape[1], H)


def kernel(ei, seq, table, W1, b1, W2, b2):
    return _run(ei, seq, table, W1, b1, W2, b2)


# speedup vs baseline: 30.0369x; 1.3563x over previous
import functools

import jax
import jax.numpy as jnp
from jax import lax
from jax.experimental import pallas as pl
from jax.experimental.pallas import tpu as pltpu
from jax.experimental.pallas import tpu_sc as plsc

N = 10000
H = 128
OFF = 2
E = 320000
NC, NS = 2, 16
NW = NC * NS
NPAD = 10240
RPT = NPAD // NS
CH = 128
EPT = 10240
K = EPT // CH
EPAD = EPT * NW
SPT = 8192 // NW
HH = 32


def _fill2d(ref, rows, val):
    cols = ref.shape[1] // 16
    v = jnp.full((16,), val, jnp.float32)

    def body(i, _):
        for c in range(cols):
            ref[i, pl.ds(c * 16, 16)] = v
        return 0

    lax.fori_loop(0, rows, body, 0)


@functools.cache
def _sc_kernels():
    mesh = plsc.VectorSubcoreMesh(
        core_axis_name="c", subcore_axis_name="s",
        num_cores=NC, num_subcores=NS)

    @functools.partial(
        pl.kernel,
        out_type=jax.ShapeDtypeStruct((NW, NPAD), jnp.float32),
        mesh=mesh,
        scratch_types=[
            pltpu.VMEM((K, CH), jnp.int32),
            pltpu.VMEM((NPAD,), jnp.float32),
        ],
        compiler_params=pltpu.CompilerParams(needs_layout_passes=False),
    )
    def deg_kernel(dst_hbm, deg_hbm, dst_v, hist):
        c = lax.axis_index("c")
        s = lax.axis_index("s")
        wid = s * NC + c
        zero = jnp.zeros((16,), jnp.float32)
        one = jnp.ones((16,), jnp.float32)

        def zbody(i, _):
            hist[pl.ds(i * 16, 16)] = zero
            return 0

        lax.fori_loop(0, NPAD // 16, zbody, 0)
        pltpu.sync_copy(dst_hbm.at[wid], dst_v)

        def body(j, _):
            for c8 in range(CH // 16):
                idx = dst_v[j, pl.ds(c8 * 16, 16)]
                plsc.addupdate_scatter(hist, [idx], one)
            return 0

        lax.fori_loop(0, K, body, 0)
        pltpu.sync_copy(hist, deg_hbm.at[wid])

    NF = H // HH

    @functools.partial(
        pl.kernel,
        out_type=jax.ShapeDtypeStruct((NC, NPAD, H), jnp.float32),
        mesh=mesh,
        scratch_types=[
            pltpu.VMEM((K, CH), jnp.int32),
            pltpu.VMEM((K, CH), jnp.int32),
            pltpu.VMEM((8, CH, HH), jnp.float32),
            pltpu.VMEM((CH, HH), jnp.float32),
            pltpu.VMEM_SHARED((NPAD, HH), jnp.float32),
            [pltpu.SemaphoreType.DMA] * 8,
            [pltpu.SemaphoreType.DMA] * 8,
        ],
        compiler_params=pltpu.CompilerParams(use_tc_tiling_on_sc=False),
    )
    def agg_kernel(hs_hbm, src_hbm, dst_hbm, part_hbm,
                   src_v, dst_v, bufs, zbuf, acc, gsems, ssems):
        c = lax.axis_index("c")
        s = lax.axis_index("s")
        wid = s * NC + c
        _fill2d(zbuf, CH, 0.0)
        pltpu.sync_copy(dst_hbm.at[wid], dst_v)

        NQ = K // 4

        for f in range(NF):
            half = hs_hbm
            pltpu.sync_copy(src_hbm.at[f, wid], src_v)
            for r in range(RPT // CH):
                pltpu.sync_copy(zbuf, acc.at[pl.ds(s * RPT + r * CH, CH)])
            plsc.subcore_barrier()

            for b in range(4):
                pltpu.async_copy(
                    half.at[src_v.at[b]], bufs.at[b], gsems[b])
            for b in range(4):
                pltpu.async_copy(
                    half.at[src_v.at[4 + b]], bufs.at[4 + b], gsems[4 + b])

            def body(qp, _):
                for hq in range(2):
                    q = 2 * qp + hq
                    aset = 4 * hq
                    j0 = 4 * q
                    for b in range(4):
                        j = j0 + b
                        slot = aset + b
                        pltpu.make_async_copy(
                            half.at[src_v.at[j]], bufs.at[slot],
                            gsems[slot]).wait()
                        pltpu.async_copy(
                            bufs.at[slot], acc.at[dst_v.at[j]],
                            ssems[slot], add=True)
                    @pl.when(q < NQ - 2)
                    def _():
                        for b in range(4):
                            j = j0 + b
                            slot = aset + b
                            pltpu.make_async_copy(
                                bufs.at[slot], acc.at[dst_v.at[j]],
                                ssems[slot]).wait()
                            pltpu.async_copy(
                                half.at[src_v.at[j + 8]], bufs.at[slot],
                                gsems[slot])
                return 0

            lax.fori_loop(0, NQ // 2, body, 0)
            for qq in (NQ - 2, NQ - 1):
                for b in range(4):
                    j = 4 * qq + b
                    slot = (qq % 2) * 4 + b
                    pltpu.make_async_copy(
                        bufs.at[slot], acc.at[dst_v.at[j]],
                        ssems[slot]).wait()
            plsc.subcore_barrier()
            pltpu.sync_copy(acc.at[pl.ds(s * RPT, RPT)],
                            part_hbm.at[c, pl.ds(s * RPT, RPT),
                                        pl.ds(f * HH, HH)])

    @functools.partial(
        pl.kernel,
        out_type=jax.ShapeDtypeStruct((8192, H), jnp.float32),
        mesh=mesh,
        scratch_types=[
            pltpu.VMEM((SPT // CH, CH), jnp.int32),
            pltpu.VMEM((CH, H), jnp.float32),
            pltpu.VMEM((CH, H), jnp.float32),
            pltpu.SemaphoreType.DMA,
            pltpu.SemaphoreType.DMA,
        ],
    )
    def seq_kernel(ztab_hbm, idx_hbm, out_hbm,
                   idx_v, buf_a, buf_b, sem_a, sem_b):
        c = lax.axis_index("c")
        s = lax.axis_index("s")
        wid = s * NC + c
        pltpu.sync_copy(idx_hbm.at[wid], idx_v)
        pltpu.async_copy(ztab_hbm.at[idx_v.at[0]], buf_a, sem_a)
        pltpu.async_copy(ztab_hbm.at[idx_v.at[1]], buf_b, sem_b)
        pltpu.make_async_copy(ztab_hbm.at[idx_v.at[0]], buf_a, sem_a).wait()
        pltpu.sync_copy(buf_a, out_hbm.at[pl.ds(wid * SPT, CH)])
        pltpu.make_async_copy(ztab_hbm.at[idx_v.at[1]], buf_b, sem_b).wait()
        pltpu.sync_copy(buf_b, out_hbm.at[pl.ds(wid * SPT + CH, CH)])

    return deg_kernel, agg_kernel, seq_kernel


RB = 1024
GRID = NPAD // RB


def _dinv_of(deg_col):
    return lax.rsqrt(deg_col + 1.0)


NF = H // HH


def _prep_body(x_ref, w_ref, deg_ref, out_ref):
    dinv = _dinv_of(deg_ref[...])
    h = jnp.dot(x_ref[...], w_ref[...], preferred_element_type=jnp.float32)
    out_ref[...] = h * dinv


def _mid_body(p_ref, hs_ref, deg_ref, b_ref, w_ref, a_ref, hs_out_ref):
    dinv = _dinv_of(deg_ref[...])
    agg = p_ref[0] + p_ref[1] + hs_ref[...]
    a = agg * dinv + b_ref[...]
    a_ref[...] = a
    h2 = jnp.dot(jnp.maximum(a, 0.0), w_ref[...],
                 preferred_element_type=jnp.float32)
    hs_out_ref[...] = h2 * dinv


_row_spec = pl.BlockSpec((RB, H), lambda i: (i, 0))
_deg_spec = pl.BlockSpec((RB, 1), lambda i: (i, 0))
_w_spec = pl.BlockSpec((H, H), lambda i: (0, 0))
_b_spec = pl.BlockSpec((1, H), lambda i: (0, 0))
_p_spec = pl.BlockSpec((NC, RB, H), lambda i: (0, i, 0))
_vec_out = jax.ShapeDtypeStruct((NPAD, H), jnp.float32)

_prep_call = pl.pallas_call(
    _prep_body, grid=(GRID,), out_shape=_vec_out,
    in_specs=[_row_spec, _w_spec, _deg_spec], out_specs=_row_spec)

_mid_call = pl.pallas_call(
    _mid_body, grid=(GRID,), out_shape=(_vec_out, _vec_out),
    in_specs=[_p_spec, _row_spec, _deg_spec, _b_spec, _w_spec],
    out_specs=(_row_spec, _row_spec))


@jax.jit
def _run(ei, seq, table, W1, b1, W2, b2):
    deg_kernel, agg_kernel, seq_kernel = _sc_kernels()
    src = ei[0].astype(jnp.int32)
    dst = ei[1].astype(jnp.int32)
    pad = EPAD - E
    spread = N + jnp.arange(pad, dtype=jnp.int32) % (NPAD - N)
    src_p = jnp.concatenate([src, spread])
    dst_p = jnp.concatenate([dst, spread])
    src_p = src_p.reshape(NW, K, CH)
    dst_p = dst_p.reshape(NW, K, CH)
    src4 = (src_p * NF)[None] + jnp.arange(NF, dtype=jnp.int32).reshape(
        NF, 1, 1, 1)

    x_pad = jnp.concatenate(
        [table[:N], jnp.zeros((NPAD - N, H), jnp.float32)])

    deg32 = deg_kernel(dst_p)
    deg2 = deg32.sum(axis=0).reshape(NPAD, 1)

    bstack = jnp.stack([b1.reshape(1, H), b2.reshape(1, H)])

    hs = _prep_call(x_pad, W1, deg2)

    p1 = agg_kernel(hs.reshape(NPAD * NF, HH), src4, dst_p)
    _, hs2 = _mid_call(p1, hs, deg2, bstack[0], W2)
    p2 = agg_kernel(hs2.reshape(NPAD * NF, HH), src4, dst_p)
    z, _ = _mid_call(p2, hs2, deg2, bstack[1], W2)

    ztab = jnp.concatenate([z[:N], table[N:N + OFF]])
    sidx = seq.astype(jnp.int32)
    idx = jnp.where(sidx >= 0, sidx, sidx + OFF + N)
    idx = idx.reshape(NW, SPT // CH, CH)
    embs = seq_kernel(ztab, idx)
    return embs.reshape(seq.shape[0], seq.shape[1], H)


def kernel(ei, seq, table, W1, b1, W2, b2):
    return _run(ei, seq, table, W1, b1, W2, b2)


# speedup vs baseline: 30.4802x; 1.0148x over previous
import functools

import jax
import jax.numpy as jnp
from jax import lax
from jax.experimental import pallas as pl
from jax.experimental.pallas import tpu as pltpu
from jax.experimental.pallas import tpu_sc as plsc

N = 10000
H = 128
OFF = 2
E = 320000
NC, NS = 2, 16
NW = NC * NS
NPAD = 10240
RPT = NPAD // NS
CH = 128
EPT = 10240
K = EPT // CH
EPAD = EPT * NW
SPT = 8192 // NW
HH = 64
AC = 64
AK = EPT // AC


def _fill2d(ref, rows, val):
    cols = ref.shape[1] // 16
    v = jnp.full((16,), val, jnp.float32)

    def body(i, _):
        for c in range(cols):
            ref[i, pl.ds(c * 16, 16)] = v
        return 0

    lax.fori_loop(0, rows, body, 0)


@functools.cache
def _sc_kernels():
    mesh = plsc.VectorSubcoreMesh(
        core_axis_name="c", subcore_axis_name="s",
        num_cores=NC, num_subcores=NS)

    @functools.partial(
        pl.kernel,
        out_type=jax.ShapeDtypeStruct((NW, NPAD), jnp.float32),
        mesh=mesh,
        scratch_types=[
            pltpu.VMEM((K, CH), jnp.int32),
            pltpu.VMEM((NPAD,), jnp.float32),
        ],
        compiler_params=pltpu.CompilerParams(needs_layout_passes=False),
    )
    def deg_kernel(dst_hbm, deg_hbm, dst_v, hist):
        c = lax.axis_index("c")
        s = lax.axis_index("s")
        wid = s * NC + c
        zero = jnp.zeros((16,), jnp.float32)
        one = jnp.ones((16,), jnp.float32)

        def zbody(i, _):
            hist[pl.ds(i * 16, 16)] = zero
            return 0

        lax.fori_loop(0, NPAD // 16, zbody, 0)
        pltpu.sync_copy(dst_hbm.at[wid], dst_v)

        def body(j, _):
            for c8 in range(CH // 16):
                idx = dst_v[j, pl.ds(c8 * 16, 16)]
                plsc.addupdate_scatter(hist, [idx], one)
            return 0

        lax.fori_loop(0, K, body, 0)
        pltpu.sync_copy(hist, deg_hbm.at[wid])

    NF = H // HH

    @functools.partial(
        pl.kernel,
        out_type=jax.ShapeDtypeStruct((NC, NPAD, H), jnp.float32),
        mesh=mesh,
        scratch_types=[
            pltpu.VMEM((AK, AC), jnp.int32),
            pltpu.VMEM((AK, AC), jnp.int32),
            pltpu.VMEM((8, AC, HH), jnp.float32),
            pltpu.VMEM((CH, HH), jnp.float32),
            pltpu.VMEM_SHARED((NPAD, HH), jnp.float32),
            [pltpu.SemaphoreType.DMA] * 8,
            [pltpu.SemaphoreType.DMA] * 8,
        ],
        compiler_params=pltpu.CompilerParams(use_tc_tiling_on_sc=False),
    )
    def agg_kernel(hs_hbm, src_hbm, dst_hbm, part_hbm,
                   src_v, dst_v, bufs, zbuf, acc, gsems, ssems):
        c = lax.axis_index("c")
        s = lax.axis_index("s")
        wid = s * NC + c
        _fill2d(zbuf, CH, 0.0)
        pltpu.sync_copy(dst_hbm.at[wid], dst_v)

        NQ = AK // 4

        for f in range(NF):
            half = hs_hbm
            pltpu.sync_copy(src_hbm.at[f, wid], src_v)
            for r in range(RPT // CH):
                pltpu.sync_copy(zbuf, acc.at[pl.ds(s * RPT + r * CH, CH)])
            plsc.subcore_barrier()

            for b in range(4):
                pltpu.async_copy(
                    half.at[src_v.at[b]], bufs.at[b], gsems[b])
            for b in range(4):
                pltpu.async_copy(
                    half.at[src_v.at[4 + b]], bufs.at[4 + b], gsems[4 + b])

            def body(qp, _):
                for hq in range(2):
                    q = 2 * qp + hq
                    aset = 4 * hq
                    j0 = 4 * q
                    for b in range(4):
                        j = j0 + b
                        slot = aset + b
                        pltpu.make_async_copy(
                            half.at[src_v.at[j]], bufs.at[slot],
                            gsems[slot]).wait()
                        pltpu.async_copy(
                            bufs.at[slot], acc.at[dst_v.at[j]],
                            ssems[slot], add=True)
                    @pl.when(q < NQ - 2)
                    def _():
                        for b in range(4):
                            j = j0 + b
                            slot = aset + b
                            pltpu.make_async_copy(
                                bufs.at[slot], acc.at[dst_v.at[j]],
                                ssems[slot]).wait()
                            pltpu.async_copy(
                                half.at[src_v.at[j + 8]], bufs.at[slot],
                                gsems[slot])
                return 0

            lax.fori_loop(0, NQ // 2, body, 0)
            for qq in (NQ - 2, NQ - 1):
                for b in range(4):
                    j = 4 * qq + b
                    slot = (qq % 2) * 4 + b
                    pltpu.make_async_copy(
                        bufs.at[slot], acc.at[dst_v.at[j]],
                        ssems[slot]).wait()
            plsc.subcore_barrier()
            pltpu.sync_copy(acc.at[pl.ds(s * RPT, RPT)],
                            part_hbm.at[c, pl.ds(s * RPT, RPT),
                                        pl.ds(f * HH, HH)])

    @functools.partial(
        pl.kernel,
        out_type=jax.ShapeDtypeStruct((8192, H), jnp.float32),
        mesh=mesh,
        scratch_types=[
            pltpu.VMEM((SPT // CH, CH), jnp.int32),
            pltpu.VMEM((CH, H), jnp.float32),
            pltpu.VMEM((CH, H), jnp.float32),
            pltpu.SemaphoreType.DMA,
            pltpu.SemaphoreType.DMA,
        ],
    )
    def seq_kernel(ztab_hbm, idx_hbm, out_hbm,
                   idx_v, buf_a, buf_b, sem_a, sem_b):
        c = lax.axis_index("c")
        s = lax.axis_index("s")
        wid = s * NC + c
        pltpu.sync_copy(idx_hbm.at[wid], idx_v)
        pltpu.async_copy(ztab_hbm.at[idx_v.at[0]], buf_a, sem_a)
        pltpu.async_copy(ztab_hbm.at[idx_v.at[1]], buf_b, sem_b)
        pltpu.make_async_copy(ztab_hbm.at[idx_v.at[0]], buf_a, sem_a).wait()
        pltpu.sync_copy(buf_a, out_hbm.at[pl.ds(wid * SPT, CH)])
        pltpu.make_async_copy(ztab_hbm.at[idx_v.at[1]], buf_b, sem_b).wait()
        pltpu.sync_copy(buf_b, out_hbm.at[pl.ds(wid * SPT + CH, CH)])

    return deg_kernel, agg_kernel, seq_kernel


RB = 1024
GRID = NPAD // RB


def _dinv_of(deg_col):
    return lax.rsqrt(deg_col + 1.0)


NF = H // HH


def _prep_body(x_ref, w_ref, deg_ref, out_ref):
    dinv = _dinv_of(deg_ref[...])
    h = jnp.dot(x_ref[...], w_ref[...], preferred_element_type=jnp.float32)
    out_ref[...] = h * dinv


def _mid_body(p_ref, hs_ref, deg_ref, b_ref, w_ref, a_ref, hs_out_ref):
    dinv = _dinv_of(deg_ref[...])
    agg = p_ref[0] + p_ref[1] + hs_ref[...]
    a = agg * dinv + b_ref[...]
    a_ref[...] = a
    h2 = jnp.dot(jnp.maximum(a, 0.0), w_ref[...],
                 preferred_element_type=jnp.float32)
    hs_out_ref[...] = h2 * dinv


_row_spec = pl.BlockSpec((RB, H), lambda i: (i, 0))
_deg_spec = pl.BlockSpec((RB, 1), lambda i: (i, 0))
_w_spec = pl.BlockSpec((H, H), lambda i: (0, 0))
_b_spec = pl.BlockSpec((1, H), lambda i: (0, 0))
_p_spec = pl.BlockSpec((NC, RB, H), lambda i: (0, i, 0))
_vec_out = jax.ShapeDtypeStruct((NPAD, H), jnp.float32)

_prep_call = pl.pallas_call(
    _prep_body, grid=(GRID,), out_shape=_vec_out,
    in_specs=[_row_spec, _w_spec, _deg_spec], out_specs=_row_spec)

_mid_call = pl.pallas_call(
    _mid_body, grid=(GRID,), out_shape=(_vec_out, _vec_out),
    in_specs=[_p_spec, _row_spec, _deg_spec, _b_spec, _w_spec],
    out_specs=(_row_spec, _row_spec))


@jax.jit
def _run(ei, seq, table, W1, b1, W2, b2):
    deg_kernel, agg_kernel, seq_kernel = _sc_kernels()
    src = ei[0].astype(jnp.int32)
    dst = ei[1].astype(jnp.int32)
    pad = EPAD - E
    spread = N + jnp.arange(pad, dtype=jnp.int32) % (NPAD - N)
    src_p = jnp.concatenate([src, spread])
    dst_p = jnp.concatenate([dst, spread])
    dst_deg = dst_p.reshape(NW, K, CH)
    dst_agg = dst_p.reshape(NW, AK, AC)
    src4 = (src_p.reshape(NW, AK, AC) * NF)[None] + jnp.arange(
        NF, dtype=jnp.int32).reshape(NF, 1, 1, 1)

    x_pad = jnp.concatenate(
        [table[:N], jnp.zeros((NPAD - N, H), jnp.float32)])

    deg32 = deg_kernel(dst_deg)
    deg2 = deg32.sum(axis=0).reshape(NPAD, 1)

    bstack = jnp.stack([b1.reshape(1, H), b2.reshape(1, H)])

    hs = _prep_call(x_pad, W1, deg2)

    def _layer(i, carry):
        hs_c, _ = carry
        p = agg_kernel(hs_c.reshape(NPAD * NF, HH), src4, dst_agg)
        a, hs_n = _mid_call(p, hs_c, deg2, bstack[i], W2)
        return (hs_n, a)

    _, z = lax.fori_loop(0, 2, _layer, (hs, jnp.zeros((NPAD, H))))

    ztab = jnp.concatenate([z[:N], table[N:N + OFF]])
    sidx = seq.astype(jnp.int32)
    idx = jnp.where(sidx >= 0, sidx, sidx + OFF + N)
    idx = idx.reshape(NW, SPT // CH, CH)
    embs = seq_kernel(ztab, idx)
    return embs.reshape(seq.shape[0], seq.shape[1], H)


def kernel(ei, seq, table, W1, b1, W2, b2):
    return _run(ei, seq, table, W1, b1, W2, b2)
